# Initial kernel scaffold; baseline (speedup 1.0000x reference)
import jax
import jax.numpy as jnp
from jax.experimental import pallas as pl


def kernel(vertices, center, faces, L_indices, L_values, K_indices, K_values, total_num):
    raise NotImplementedError("write your pallas kernel here")


# speedup vs baseline: 6.3333x; 6.3333x over previous
import functools

import jax
import jax.numpy as jnp
from jax import lax
from jax.experimental import pallas as pl
from jax.experimental.pallas import tpu as pltpu
from jax.experimental.pallas import tpu_sc as plsc

V = 100000
F = 200000
NNZ = 700000

NC = 2
NS = 16
NW = NC * NS
LANES = 16

S = 2048
NSUB = 11
Q = NSUB * S
NNZ_PAD = NW * Q

VP = 100096
CHK = VP // NS


def _tc_prep_call(vertices, center, faces, one_f, one_i):
    BV = 1024
    BF = 2048
    grid = pl.cdiv(V, BV)

    def body(vert_ref, cen_ref, face_ref, onef_ref, onei_ref,
             vout_ref, fout_ref, x_ref, y_ref, z_ref):
        v = vert_ref[...] + cen_ref[...]
        vf = v * onef_ref[0, 0]
        vout_ref[...] = jnp.broadcast_to(vf[None], (4, BV, 3))
        fout_ref[...] = jnp.broadcast_to(
            face_ref[...][None] * onei_ref[0, 0], (4, BF, 3))
        x_ref[...] = v[:, 0]
        y_ref[...] = v[:, 1]
        z_ref[...] = v[:, 2]

    return pl.pallas_call(
        body,
        grid=(grid,),
        in_specs=[
            pl.BlockSpec((BV, 3), lambda i: (i, 0)),
            pl.BlockSpec((1, 3), lambda i: (0, 0)),
            pl.BlockSpec((BF, 3), lambda i: (i, 0)),
            pl.BlockSpec(memory_space=pltpu.SMEM),
            pl.BlockSpec(memory_space=pltpu.SMEM),
        ],
        out_specs=[
            pl.BlockSpec((4, BV, 3), lambda i: (0, i, 0)),
            pl.BlockSpec((4, BF, 3), lambda i: (0, i, 0)),
            pl.BlockSpec((BV,), lambda i: (i,)),
            pl.BlockSpec((BV,), lambda i: (i,)),
            pl.BlockSpec((BV,), lambda i: (i,)),
        ],
        out_shape=[
            jax.ShapeDtypeStruct((4, V, 3), jnp.float32),
            jax.ShapeDtypeStruct((4, F, 3), jnp.int32),
            jax.ShapeDtypeStruct((V,), jnp.float32),
            jax.ShapeDtypeStruct((V,), jnp.float32),
            jax.ShapeDtypeStruct((V,), jnp.float32),
        ],
    )(vertices, center, faces, one_f, one_i)


def _sc_spmv_call(x, y, z, rowL, colL, valL, rowK, colK, valK):
    mesh = plsc.VectorSubcoreMesh(core_axis_name="c", subcore_axis_name="s",
                                  num_cores=NC, num_subcores=NS)

    def body(x_hbm, y_hbm, z_hbm,
             rL_hbm, cL_hbm, vL_hbm, rK_hbm, cK_hbm, vK_hbm,
             out_hbm,
             aLx, aLy, aLz, aKx, aKy, aKz,
             col_v, row_v, val_v, gx, gy, gz, zbuf,
             sem0, sem1, sem2):
        cid = lax.axis_index("c")
        sid = lax.axis_index("s")

        def zero_body(i, _):
            zbuf[pl.ds(i * LANES, LANES)] = jnp.zeros((LANES,), jnp.float32)
            return 0
        lax.fori_loop(0, CHK // LANES, zero_body, 0)
        for acc in (aLx, aLy, aLz, aKx, aKy, aKz):
            pltpu.sync_copy(zbuf, acc.at[pl.ds(sid * CHK, CHK)])
        plsc.subcore_barrier()

        wid = cid * NS + sid

        def do_matrix(r_hbm, c_hbm, v_hbm, ax, ay, az):
            for j in range(NSUB):
                base = wid * Q + j * S
                pltpu.sync_copy(c_hbm.at[pl.ds(base, S)], col_v)
                pltpu.sync_copy(r_hbm.at[pl.ds(base, S)], row_v)
                pltpu.sync_copy(v_hbm.at[pl.ds(base, S)], val_v)
                d0 = pltpu.async_copy(x_hbm.at[col_v], gx, sem0)
                d1 = pltpu.async_copy(y_hbm.at[col_v], gy, sem1)
                d2 = pltpu.async_copy(z_hbm.at[col_v], gz, sem2)
                d0.wait()
                d1.wait()
                d2.wait()

                def mul_body(i, _):
                    sl = pl.ds(i * LANES, LANES)
                    w = val_v[sl]
                    gx[sl] = gx[sl] * w
                    gy[sl] = gy[sl] * w
                    gz[sl] = gz[sl] * w
                    return 0
                lax.fori_loop(0, S // LANES, mul_body, 0)

                pltpu.sync_copy(gx, ax.at[row_v], add=True)
                pltpu.sync_copy(gy, ay.at[row_v], add=True)
                pltpu.sync_copy(gz, az.at[row_v], add=True)

        do_matrix(rL_hbm, cL_hbm, vL_hbm, aLx, aLy, aLz)
        do_matrix(rK_hbm, cK_hbm, vK_hbm, aKx, aKy, aKz)

        plsc.subcore_barrier()

        for j, acc in enumerate((aLx, aLy, aLz, aKx, aKy, aKz)):
            off = (cid * 6 + j) * VP + sid * CHK
            pltpu.sync_copy(acc.at[pl.ds(sid * CHK, CHK)], zbuf)
            pltpu.sync_copy(zbuf, out_hbm.at[pl.ds(off, CHK)])

    kfn = pl.kernel(
        body,
        out_type=jax.ShapeDtypeStruct((NC * 6 * VP,), jnp.float32),
        mesh=mesh,
        scratch_types=[
            pltpu.VMEM_SHARED((VP,), jnp.float32),
            pltpu.VMEM_SHARED((VP,), jnp.float32),
            pltpu.VMEM_SHARED((VP,), jnp.float32),
            pltpu.VMEM_SHARED((VP,), jnp.float32),
            pltpu.VMEM_SHARED((VP,), jnp.float32),
            pltpu.VMEM_SHARED((VP,), jnp.float32),
            pltpu.VMEM((S,), jnp.int32),
            pltpu.VMEM((S,), jnp.int32),
            pltpu.VMEM((S,), jnp.float32),
            pltpu.VMEM((S,), jnp.float32),
            pltpu.VMEM((S,), jnp.float32),
            pltpu.VMEM((S,), jnp.float32),
            pltpu.VMEM((CHK,), jnp.float32),
            pltpu.SemaphoreType.DMA,
            pltpu.SemaphoreType.DMA,
            pltpu.SemaphoreType.DMA,
        ],
    )
    return kfn(x, y, z, rowL, colL, valL, rowK, colK, valK)


def _tc_reduce_call(parts):
    def body(p_ref, out_ref):
        p = p_ref[0] + p_ref[1]
        lap = p[0:3] + jnp.float32(1e-12)
        norm = jnp.sqrt(jnp.sum(lap * lap, axis=0))
        kv = p[3:6]
        ksq = jnp.sum(kv * kv, axis=0)
        out_ref[0, 0] = jnp.sum(norm) / jnp.float32(V)
        out_ref[0, 1] = jnp.sum(ksq) / jnp.float32(V)

    return pl.pallas_call(
        body,
        out_specs=pl.BlockSpec(memory_space=pltpu.SMEM),
        out_shape=jax.ShapeDtypeStruct((1, 2), jnp.float32),
    )(parts)


def kernel(vertices, center, faces, L_indices, L_values, K_indices, K_values,
           total_num):
    one_i = jnp.asarray(total_num, dtype=jnp.int32) // 4
    one_f = one_i.astype(jnp.float32)
    one_i_s = jnp.reshape(one_i, (1, 1))
    one_f_s = jnp.reshape(one_f, (1, 1))

    verts_out, faces_out, x, y, z = _tc_prep_call(
        vertices, center, faces, one_f_s, one_i_s)

    pad = NNZ_PAD - NNZ
    zi = jnp.zeros((pad,), jnp.int32)
    zf = jnp.zeros((pad,), jnp.float32)
    rowL = jnp.concatenate([L_indices[0], zi])
    colL = jnp.concatenate([L_indices[1], zi])
    valL = jnp.concatenate([L_values, zf])
    rowK = jnp.concatenate([K_indices[0], zi])
    colK = jnp.concatenate([K_indices[1], zi])
    valK = jnp.concatenate([K_values, zf])

    parts = _sc_spmv_call(x, y, z, rowL, colL, valL, rowK, colK, valK)
    losses = _tc_reduce_call(parts.reshape(NC, 6, VP))

    laplacian_loss = losses[0, 0]
    hexagon_loss = losses[0, 1]
    zero = jnp.float32(0.0)
    return (verts_out, faces_out, laplacian_loss, hexagon_loss, zero, zero)


# speedup vs baseline: 9.2664x; 1.4631x over previous
import functools

import jax
import jax.numpy as jnp
from jax import lax
from jax.experimental import pallas as pl
from jax.experimental.pallas import tpu as pltpu
from jax.experimental.pallas import tpu_sc as plsc

V = 100000
F = 200000
NNZ = 700000

NC = 2
NS = 16
NW = NC * NS
LANES = 16

S = 2736
NSUB = 8
Q = NSUB * S
NNZ_PAD = NW * Q

VP = 100096
CHK = VP // NS
CHK_LAST = V - (NS - 1) * CHK


def _tc_prep_call(vertices, center, faces, one_f, one_i):
    BV = 1024
    BF = 2048
    grid = pl.cdiv(V, BV)

    def body(vert_ref, cen_ref, face_ref, onef_ref, onei_ref,
             vout_ref, fout_ref, x_ref, y_ref, z_ref):
        v = vert_ref[...] + cen_ref[...]
        vf = v * onef_ref[0, 0]
        vout_ref[...] = jnp.broadcast_to(vf[None], (4, BV, 3))
        fout_ref[...] = jnp.broadcast_to(
            face_ref[...][None] * onei_ref[0, 0], (4, BF, 3))
        x_ref[...] = v[:, 0]
        y_ref[...] = v[:, 1]
        z_ref[...] = v[:, 2]

    return pl.pallas_call(
        body,
        grid=(grid,),
        in_specs=[
            pl.BlockSpec((BV, 3), lambda i: (i, 0)),
            pl.BlockSpec((1, 3), lambda i: (0, 0)),
            pl.BlockSpec((BF, 3), lambda i: (i, 0)),
            pl.BlockSpec(memory_space=pltpu.SMEM),
            pl.BlockSpec(memory_space=pltpu.SMEM),
        ],
        out_specs=[
            pl.BlockSpec((4, BV, 3), lambda i: (0, i, 0)),
            pl.BlockSpec((4, BF, 3), lambda i: (0, i, 0)),
            pl.BlockSpec((BV,), lambda i: (i,)),
            pl.BlockSpec((BV,), lambda i: (i,)),
            pl.BlockSpec((BV,), lambda i: (i,)),
        ],
        out_shape=[
            jax.ShapeDtypeStruct((4, V, 3), jnp.float32),
            jax.ShapeDtypeStruct((4, F, 3), jnp.int32),
            jax.ShapeDtypeStruct((V,), jnp.float32),
            jax.ShapeDtypeStruct((V,), jnp.float32),
            jax.ShapeDtypeStruct((V,), jnp.float32),
        ],
    )(vertices, center, faces, one_f, one_i)


def _sc_spmv_call(x, y, z, rowL, colL, valL, rowK, colK, valK):
    mesh = plsc.VectorSubcoreMesh(core_axis_name="c", subcore_axis_name="s",
                                  num_cores=NC, num_subcores=NS)

    def body(x_hbm, y_hbm, z_hbm,
             rL_hbm, cL_hbm, vL_hbm, rK_hbm, cK_hbm, vK_hbm,
             out_hbm,
             aLx, aLy, aLz, aKx, aKy, aKz,
             sx, sy, sz,
             col_v, row_v, val_v, gx, gy, gz, zbuf,
             sem0, sem1, sem2):
        cid = lax.axis_index("c")
        sid = lax.axis_index("s")

        def zero_body(i, _):
            zbuf[pl.ds(i * LANES, LANES)] = jnp.zeros((LANES,), jnp.float32)
            return 0
        lax.fori_loop(0, CHK // LANES, zero_body, 0)
        for acc in (aLx, aLy, aLz, aKx, aKy, aKz):
            pltpu.sync_copy(zbuf, acc.at[pl.ds(sid * CHK, CHK)])

        n = jnp.where(sid == NS - 1, CHK_LAST, CHK)
        for src_hbm, ssrc in ((x_hbm, sx), (y_hbm, sy), (z_hbm, sz)):
            pltpu.sync_copy(src_hbm.at[pl.ds(sid * CHK, n)],
                            zbuf.at[pl.ds(0, n)])
            pltpu.sync_copy(zbuf.at[pl.ds(0, n)],
                            ssrc.at[pl.ds(sid * CHK, n)])
        plsc.subcore_barrier()

        wid = cid * NS + sid

        def do_matrix(r_hbm, c_hbm, v_hbm, ax, ay, az):
            for j in range(NSUB):
                base = wid * Q + j * S
                pltpu.sync_copy(c_hbm.at[pl.ds(base, S)], col_v)
                pltpu.sync_copy(r_hbm.at[pl.ds(base, S)], row_v)
                pltpu.sync_copy(v_hbm.at[pl.ds(base, S)], val_v)
                d0 = pltpu.async_copy(sx.at[col_v], gx, sem0)
                d1 = pltpu.async_copy(sy.at[col_v], gy, sem1)
                d2 = pltpu.async_copy(sz.at[col_v], gz, sem2)
                d0.wait()
                d1.wait()
                d2.wait()

                def mul_body(i, _):
                    sl = pl.ds(i * LANES, LANES)
                    w = val_v[sl]
                    gx[sl] = gx[sl] * w
                    gy[sl] = gy[sl] * w
                    gz[sl] = gz[sl] * w
                    return 0
                lax.fori_loop(0, S // LANES, mul_body, 0)

                pltpu.sync_copy(gx, ax.at[row_v], add=True)
                pltpu.sync_copy(gy, ay.at[row_v], add=True)
                pltpu.sync_copy(gz, az.at[row_v], add=True)

        do_matrix(rL_hbm, cL_hbm, vL_hbm, aLx, aLy, aLz)
        do_matrix(rK_hbm, cK_hbm, vK_hbm, aKx, aKy, aKz)

        plsc.subcore_barrier()

        for j, acc in enumerate((aLx, aLy, aLz, aKx, aKy, aKz)):
            off = (cid * 6 + j) * VP + sid * CHK
            pltpu.sync_copy(acc.at[pl.ds(sid * CHK, CHK)], zbuf)
            pltpu.sync_copy(zbuf, out_hbm.at[pl.ds(off, CHK)])

    kfn = pl.kernel(
        body,
        out_type=jax.ShapeDtypeStruct((NC * 6 * VP,), jnp.float32),
        mesh=mesh,
        scratch_types=[
            pltpu.VMEM_SHARED((VP,), jnp.float32),
            pltpu.VMEM_SHARED((VP,), jnp.float32),
            pltpu.VMEM_SHARED((VP,), jnp.float32),
            pltpu.VMEM_SHARED((VP,), jnp.float32),
            pltpu.VMEM_SHARED((VP,), jnp.float32),
            pltpu.VMEM_SHARED((VP,), jnp.float32),
            pltpu.VMEM_SHARED((VP,), jnp.float32),
            pltpu.VMEM_SHARED((VP,), jnp.float32),
            pltpu.VMEM_SHARED((VP,), jnp.float32),
            pltpu.VMEM((S,), jnp.int32),
            pltpu.VMEM((S,), jnp.int32),
            pltpu.VMEM((S,), jnp.float32),
            pltpu.VMEM((S,), jnp.float32),
            pltpu.VMEM((S,), jnp.float32),
            pltpu.VMEM((S,), jnp.float32),
            pltpu.VMEM((CHK,), jnp.float32),
            pltpu.SemaphoreType.DMA,
            pltpu.SemaphoreType.DMA,
            pltpu.SemaphoreType.DMA,
        ],
    )
    return kfn(x, y, z, rowL, colL, valL, rowK, colK, valK)


def _tc_reduce_call(parts):
    def body(p_ref, out_ref):
        p = p_ref[0] + p_ref[1]
        lap = p[0:3] + jnp.float32(1e-12)
        norm = jnp.sqrt(jnp.sum(lap * lap, axis=0))
        kv = p[3:6]
        ksq = jnp.sum(kv * kv, axis=0)
        out_ref[0, 0] = jnp.sum(norm) / jnp.float32(V)
        out_ref[0, 1] = jnp.sum(ksq) / jnp.float32(V)

    return pl.pallas_call(
        body,
        out_specs=pl.BlockSpec(memory_space=pltpu.SMEM),
        out_shape=jax.ShapeDtypeStruct((1, 2), jnp.float32),
    )(parts)


def kernel(vertices, center, faces, L_indices, L_values, K_indices, K_values,
           total_num):
    one_i = jnp.asarray(total_num, dtype=jnp.int32) // 4
    one_f = one_i.astype(jnp.float32)
    one_i_s = jnp.reshape(one_i, (1, 1))
    one_f_s = jnp.reshape(one_f, (1, 1))

    verts_out, faces_out, x, y, z = _tc_prep_call(
        vertices, center, faces, one_f_s, one_i_s)

    pad = NNZ_PAD - NNZ
    zi = jnp.zeros((pad,), jnp.int32)
    zf = jnp.zeros((pad,), jnp.float32)
    rowL = jnp.concatenate([L_indices[0], zi])
    colL = jnp.concatenate([L_indices[1], zi])
    valL = jnp.concatenate([L_values, zf])
    rowK = jnp.concatenate([K_indices[0], zi])
    colK = jnp.concatenate([K_indices[1], zi])
    valK = jnp.concatenate([K_values, zf])

    parts = _sc_spmv_call(x, y, z, rowL, colL, valL, rowK, colK, valK)
    losses = _tc_reduce_call(parts.reshape(NC, 6, VP))

    laplacian_loss = losses[0, 0]
    hexagon_loss = losses[0, 1]
    zero = jnp.float32(0.0)
    return (verts_out, faces_out, laplacian_loss, hexagon_loss, zero, zero)
